# baseline (device time: 99621 ns/iter reference)
import os

import jax
import jax.numpy as jnp
from jax import lax
from jax.experimental import pallas as pl
from jax.experimental.pallas import tpu as pltpu

_NO_COMM = os.environ.get("KERNEL_NO_COMM") == "1"

N_DEV = 8
SQ = 256
CHUNK = SQ // N_DEV
SKV_LOCAL = 4096
NSEG = 8
SEG = SKV_LOCAL // NSEG
HQ = 8
DH = 128
D = 1024
BLK = 64
SCALE = 0.08838834764831843
NEG = -1e9


def kernel(x, Wq, K_ext, V_ext, Wo):
    def body(x_ref, wq_ref, k_ref, v_ref, wo_ref, out_ref,
             q_all, local_o, local_l, rs_o, rs_l, khead, vhead,
             hd_sems,
             rs_send_o, rs_send_l, rs_recv_o, rs_recv_l,
             ag_send, ag_recv):
        my = lax.axis_index("i")
        g = pl.program_id(0)

        @pl.when(g == 0)
        def _():
            barrier = pltpu.get_barrier_semaphore()
            for p in range(N_DEV):
                pl.semaphore_signal(
                    barrier, inc=1,
                    device_id=(p,), device_id_type=pl.DeviceIdType.MESH,
                )
            pl.semaphore_wait(barrier, N_DEV)

            q_all[...] = (jnp.dot(
                x_ref[0].astype(jnp.bfloat16),
                wq_ref[...].astype(jnp.bfloat16),
                preferred_element_type=jnp.float32,
            ) * SCALE).astype(jnp.bfloat16)
            local_l[...] = jnp.zeros((SQ, HQ), jnp.float32)

        qb = lax.broadcasted_iota(jnp.int32, (SQ, 1), 0) // BLK
        kb = lax.broadcasted_iota(jnp.int32, (1, SEG), 1) // BLK
        kb = kb + my * (SKV_LOCAL // BLK) + g * (SEG // BLK)
        mask = (qb == kb) | (kb == 0) | ((qb + kb) % 3 == 0)

        copies = []
        for h in range(HQ):
            ck = pltpu.make_async_copy(
                k_ref.at[0, :, h, :], khead.at[h], hd_sems.at[0, h])
            cv = pltpu.make_async_copy(
                v_ref.at[0, :, h, :], vhead.at[h], hd_sems.at[1, h])
            ck.start()
            cv.start()
            copies.append((ck, cv))

        l_cols = lax.broadcasted_iota(jnp.int32, (SQ, HQ), 1)
        l_acc = local_l[...]
        for h in range(HQ):
            copies[h][0].wait()
            q_h = q_all[:, h * DH:(h + 1) * DH]
            k_h = khead[h].astype(jnp.bfloat16)
            s = lax.dot_general(
                q_h, k_h, (((1,), (1,)), ((), ())),
                preferred_element_type=jnp.float32,
            )
            s = jnp.where(mask, s, NEG)
            w = jnp.exp(s.astype(jnp.bfloat16))
            l_h = jnp.sum(w.astype(jnp.float32), axis=1)
            l_acc = jnp.where(l_cols == h, l_acc + l_h[:, None], l_acc)
            copies[h][1].wait()
            v_h = vhead[h].astype(jnp.bfloat16)
            o_h = lax.dot_general(
                w, v_h, (((1,), (0,)), ((), ())),
                preferred_element_type=jnp.float32,
            )

            @pl.when(g == 0)
            def _():
                local_o[h] = o_h

            @pl.when(g != 0)
            def _():
                local_o[h] = local_o[h] + o_h
        local_l[...] = l_acc

        @pl.when(g == NSEG - 1)
        def _():
            for p in range(N_DEV) if not _NO_COMM else []:
                @pl.when(my != p)
                def _():
                    d_o = pltpu.make_async_remote_copy(
                        src_ref=local_o.at[:, p * CHUNK:(p + 1) * CHUNK, :],
                        dst_ref=rs_o.at[my],
                        send_sem=rs_send_o.at[p], recv_sem=rs_recv_o.at[my],
                        device_id=(p,), device_id_type=pl.DeviceIdType.MESH,
                    )
                    d_l = pltpu.make_async_remote_copy(
                        src_ref=local_l.at[p * CHUNK:(p + 1) * CHUNK, :],
                        dst_ref=rs_l.at[my],
                        send_sem=rs_send_l.at[p], recv_sem=rs_recv_l.at[my],
                        device_id=(p,), device_id_type=pl.DeviceIdType.MESH,
                    )
                    d_o.start()
                    d_l.start()

            rs_o[my] = local_o[:, pl.ds(my * CHUNK, CHUNK), :]
            rs_l[my] = local_l[pl.ds(my * CHUNK, CHUNK), :]

            for p in range(N_DEV) if not _NO_COMM else []:
                @pl.when(my != p)
                def _():
                    r_o = pltpu.make_async_remote_copy(
                        src_ref=rs_o.at[p], dst_ref=rs_o.at[p],
                        send_sem=rs_send_o.at[p], recv_sem=rs_recv_o.at[p],
                        device_id=(p,), device_id_type=pl.DeviceIdType.MESH,
                    )
                    r_l = pltpu.make_async_remote_copy(
                        src_ref=rs_l.at[p], dst_ref=rs_l.at[p],
                        send_sem=rs_send_l.at[p], recv_sem=rs_recv_l.at[p],
                        device_id=(p,), device_id_type=pl.DeviceIdType.MESH,
                    )
                    r_o.wait_recv()
                    r_l.wait_recv()

            l_sum = jnp.sum(rs_l[...], axis=0)
            l_t = jnp.transpose(l_sum, (1, 0))
            o_acc = rs_o[0]
            for p in range(1, N_DEV):
                o_acc = o_acc + rs_o[p]
            ctx = o_acc / l_t[:, :, None]

            ctx2d = jnp.concatenate([ctx[i] for i in range(HQ)], axis=1)
            out_chunk = jnp.dot(
                ctx2d.astype(jnp.bfloat16),
                wo_ref[...].astype(jnp.bfloat16),
                preferred_element_type=jnp.float32,
            )
            out_ref[0, pl.ds(my * CHUNK, CHUNK), :] = out_chunk

            for p in range(N_DEV) if not _NO_COMM else []:
                @pl.when(my != p)
                def _():
                    d_g = pltpu.make_async_remote_copy(
                        src_ref=out_ref.at[0, pl.ds(my * CHUNK, CHUNK), :],
                        dst_ref=out_ref.at[0, pl.ds(my * CHUNK, CHUNK), :],
                        send_sem=ag_send.at[p], recv_sem=ag_recv.at[my],
                        device_id=(p,), device_id_type=pl.DeviceIdType.MESH,
                    )
                    d_g.start()
            for p in range(N_DEV) if not _NO_COMM else []:
                @pl.when(my != p)
                def _():
                    r_g = pltpu.make_async_remote_copy(
                        src_ref=out_ref.at[0, pl.ds(p * CHUNK, CHUNK), :],
                        dst_ref=out_ref.at[0, pl.ds(p * CHUNK, CHUNK), :],
                        send_sem=ag_send.at[p], recv_sem=ag_recv.at[p],
                        device_id=(p,), device_id_type=pl.DeviceIdType.MESH,
                    )
                    r_g.wait_recv()

            for p in range(N_DEV) if not _NO_COMM else []:
                @pl.when(my != p)
                def _():
                    w_o = pltpu.make_async_remote_copy(
                        src_ref=local_o.at[:, p * CHUNK:(p + 1) * CHUNK, :],
                        dst_ref=rs_o.at[my],
                        send_sem=rs_send_o.at[p], recv_sem=rs_recv_o.at[my],
                        device_id=(p,), device_id_type=pl.DeviceIdType.MESH,
                    )
                    w_l = pltpu.make_async_remote_copy(
                        src_ref=local_l.at[p * CHUNK:(p + 1) * CHUNK, :],
                        dst_ref=rs_l.at[my],
                        send_sem=rs_send_l.at[p], recv_sem=rs_recv_l.at[my],
                        device_id=(p,), device_id_type=pl.DeviceIdType.MESH,
                    )
                    w_g = pltpu.make_async_remote_copy(
                        src_ref=out_ref.at[0, pl.ds(my * CHUNK, CHUNK), :],
                        dst_ref=out_ref.at[0, pl.ds(my * CHUNK, CHUNK), :],
                        send_sem=ag_send.at[p], recv_sem=ag_recv.at[my],
                        device_id=(p,), device_id_type=pl.DeviceIdType.MESH,
                    )
                    w_o.wait_send()
                    w_l.wait_send()
                    w_g.wait_send()

    return pl.pallas_call(
        body,
        grid=(NSEG,),
        out_shape=jax.ShapeDtypeStruct((1, SQ, D), jnp.float32),
        in_specs=[
            pl.BlockSpec((1, SQ, D), lambda g: (0, 0, 0)),
            pl.BlockSpec((D, D), lambda g: (0, 0)),
            pl.BlockSpec((1, SEG, HQ, DH), lambda g: (0, g, 0, 0)),
            pl.BlockSpec((1, SEG, HQ, DH), lambda g: (0, g, 0, 0)),
            pl.BlockSpec((D, D), lambda g: (0, 0)),
        ],
        out_specs=pl.BlockSpec((1, SQ, D), lambda g: (0, 0, 0)),
        scratch_shapes=[
            pltpu.VMEM((SQ, D), jnp.bfloat16),
            pltpu.VMEM((HQ, SQ, DH), jnp.float32),
            pltpu.VMEM((SQ, HQ), jnp.float32),
            pltpu.VMEM((N_DEV, HQ, CHUNK, DH), jnp.float32),
            pltpu.VMEM((N_DEV, CHUNK, HQ), jnp.float32),
            pltpu.VMEM((HQ, SEG, DH), jnp.float32),
            pltpu.VMEM((HQ, SEG, DH), jnp.float32),
            pltpu.SemaphoreType.DMA((2, HQ)),
            pltpu.SemaphoreType.DMA((N_DEV,)),
            pltpu.SemaphoreType.DMA((N_DEV,)),
            pltpu.SemaphoreType.DMA((N_DEV,)),
            pltpu.SemaphoreType.DMA((N_DEV,)),
            pltpu.SemaphoreType.DMA((N_DEV,)),
            pltpu.SemaphoreType.DMA((N_DEV,)),
        ],
        compiler_params=pltpu.CompilerParams(
            collective_id=0, vmem_limit_bytes=64 * 1024 * 1024,
        ),
    )(x, Wq, K_ext, V_ext, Wo)


# device time: 92734 ns/iter; 1.0743x vs baseline; 1.0743x over previous
import os

import jax
import jax.numpy as jnp
from jax import lax
from jax.experimental import pallas as pl
from jax.experimental.pallas import tpu as pltpu

_NO_COMM = os.environ.get("KERNEL_NO_COMM") == "1"

N_DEV = 8
SQ = 256
CHUNK = SQ // N_DEV
SKV_LOCAL = 4096
HQ = 8
DH = 128
D = 1024
BLK = 64
SCALE = 0.08838834764831843
NEG = -1e9


def kernel(x, Wq, K_ext, V_ext, Wo):
    def body(x_ref, wq_ref, k_ref, v_ref, wo_ref, out_ref,
             local_o, local_l, rs_o, rs_l,
             rs_send_o, rs_send_l, rs_recv_o, rs_recv_l,
             ag_send, ag_recv):
        my = lax.axis_index("i")
        h = pl.program_id(0)

        @pl.when(h == 0)
        def _():
            barrier = pltpu.get_barrier_semaphore()
            for p in range(N_DEV):
                pl.semaphore_signal(
                    barrier, inc=1,
                    device_id=(p,), device_id_type=pl.DeviceIdType.MESH,
                )
            pl.semaphore_wait(barrier, N_DEV)

        q_h = (jnp.dot(
            x_ref[0].astype(jnp.bfloat16),
            wq_ref[...].astype(jnp.bfloat16),
            preferred_element_type=jnp.float32,
        ) * SCALE).astype(jnp.bfloat16)

        qb = lax.broadcasted_iota(jnp.int32, (SQ, 1), 0) // BLK
        kb = lax.broadcasted_iota(jnp.int32, (1, SKV_LOCAL), 1) // BLK
        kb = kb + my * (SKV_LOCAL // BLK)
        mask = (qb == kb) | (kb == 0) | ((qb + kb) % 3 == 0)

        s = lax.dot_general(
            q_h, k_ref[0], (((1,), (1,)), ((), ())),
            preferred_element_type=jnp.float32,
        )
        s = jnp.where(mask, s, NEG)
        w = jnp.exp(s.astype(jnp.bfloat16))
        l_h = jnp.sum(w.astype(jnp.float32), axis=1)
        o_h = lax.dot_general(
            w, v_ref[0], (((1,), (0,)), ((), ())),
            preferred_element_type=jnp.float32,
        )
        local_o[h] = o_h
        col = lax.broadcasted_iota(jnp.int32, (SQ, HQ), 1)
        local_l[...] = jnp.where(col == h, l_h[:, None], local_l[...])

        @pl.when(h == HQ - 1)
        def _():
            for p in range(N_DEV) if not _NO_COMM else []:
                @pl.when(my != p)
                def _():
                    d_o = pltpu.make_async_remote_copy(
                        src_ref=local_o.at[:, p * CHUNK:(p + 1) * CHUNK, :],
                        dst_ref=rs_o.at[my],
                        send_sem=rs_send_o.at[p], recv_sem=rs_recv_o.at[my],
                        device_id=(p,), device_id_type=pl.DeviceIdType.MESH,
                    )
                    d_l = pltpu.make_async_remote_copy(
                        src_ref=local_l.at[p * CHUNK:(p + 1) * CHUNK, :],
                        dst_ref=rs_l.at[my],
                        send_sem=rs_send_l.at[p], recv_sem=rs_recv_l.at[my],
                        device_id=(p,), device_id_type=pl.DeviceIdType.MESH,
                    )
                    d_o.start()
                    d_l.start()

            rs_o[my] = local_o[:, pl.ds(my * CHUNK, CHUNK), :]
            rs_l[my] = local_l[pl.ds(my * CHUNK, CHUNK), :]

            for p in range(N_DEV) if not _NO_COMM else []:
                @pl.when(my != p)
                def _():
                    r_o = pltpu.make_async_remote_copy(
                        src_ref=rs_o.at[p], dst_ref=rs_o.at[p],
                        send_sem=rs_send_o.at[p], recv_sem=rs_recv_o.at[p],
                        device_id=(p,), device_id_type=pl.DeviceIdType.MESH,
                    )
                    r_l = pltpu.make_async_remote_copy(
                        src_ref=rs_l.at[p], dst_ref=rs_l.at[p],
                        send_sem=rs_send_l.at[p], recv_sem=rs_recv_l.at[p],
                        device_id=(p,), device_id_type=pl.DeviceIdType.MESH,
                    )
                    r_o.wait_recv()
                    r_l.wait_recv()

            l_sum = jnp.sum(rs_l[...], axis=0)
            l_t = jnp.transpose(l_sum, (1, 0))
            o_acc = rs_o[0]
            for p in range(1, N_DEV):
                o_acc = o_acc + rs_o[p]
            ctx = o_acc / l_t[:, :, None]

            ctx2d = jnp.concatenate([ctx[i] for i in range(HQ)], axis=1)
            out_chunk = jnp.dot(
                ctx2d.astype(jnp.bfloat16),
                wo_ref[...].astype(jnp.bfloat16),
                preferred_element_type=jnp.float32,
            )
            out_ref[0, pl.ds(my * CHUNK, CHUNK), :] = out_chunk

            for p in range(N_DEV) if not _NO_COMM else []:
                @pl.when(my != p)
                def _():
                    d_g = pltpu.make_async_remote_copy(
                        src_ref=out_ref.at[0, pl.ds(my * CHUNK, CHUNK), :],
                        dst_ref=out_ref.at[0, pl.ds(my * CHUNK, CHUNK), :],
                        send_sem=ag_send.at[p], recv_sem=ag_recv.at[my],
                        device_id=(p,), device_id_type=pl.DeviceIdType.MESH,
                    )
                    d_g.start()
            for p in range(N_DEV) if not _NO_COMM else []:
                @pl.when(my != p)
                def _():
                    r_g = pltpu.make_async_remote_copy(
                        src_ref=out_ref.at[0, pl.ds(p * CHUNK, CHUNK), :],
                        dst_ref=out_ref.at[0, pl.ds(p * CHUNK, CHUNK), :],
                        send_sem=ag_send.at[p], recv_sem=ag_recv.at[p],
                        device_id=(p,), device_id_type=pl.DeviceIdType.MESH,
                    )
                    r_g.wait_recv()

            for p in range(N_DEV) if not _NO_COMM else []:
                @pl.when(my != p)
                def _():
                    w_o = pltpu.make_async_remote_copy(
                        src_ref=local_o.at[:, p * CHUNK:(p + 1) * CHUNK, :],
                        dst_ref=rs_o.at[my],
                        send_sem=rs_send_o.at[p], recv_sem=rs_recv_o.at[my],
                        device_id=(p,), device_id_type=pl.DeviceIdType.MESH,
                    )
                    w_l = pltpu.make_async_remote_copy(
                        src_ref=local_l.at[p * CHUNK:(p + 1) * CHUNK, :],
                        dst_ref=rs_l.at[my],
                        send_sem=rs_send_l.at[p], recv_sem=rs_recv_l.at[my],
                        device_id=(p,), device_id_type=pl.DeviceIdType.MESH,
                    )
                    w_g = pltpu.make_async_remote_copy(
                        src_ref=out_ref.at[0, pl.ds(my * CHUNK, CHUNK), :],
                        dst_ref=out_ref.at[0, pl.ds(my * CHUNK, CHUNK), :],
                        send_sem=ag_send.at[p], recv_sem=ag_recv.at[my],
                        device_id=(p,), device_id_type=pl.DeviceIdType.MESH,
                    )
                    w_o.wait_send()
                    w_l.wait_send()
                    w_g.wait_send()

    return pl.pallas_call(
        body,
        grid=(HQ,),
        out_shape=jax.ShapeDtypeStruct((1, SQ, D), jnp.float32),
        in_specs=[
            pl.BlockSpec((1, SQ, D), lambda h: (0, 0, 0)),
            pl.BlockSpec((D, DH), lambda h: (0, h)),
            pl.BlockSpec((1, SKV_LOCAL, DH), lambda h: (0, 0, h)),
            pl.BlockSpec((1, SKV_LOCAL, DH), lambda h: (0, 0, h)),
            pl.BlockSpec((D, D), lambda h: (0, 0)),
        ],
        out_specs=pl.BlockSpec((1, SQ, D), lambda h: (0, 0, 0)),
        scratch_shapes=[
            pltpu.VMEM((HQ, SQ, DH), jnp.float32),
            pltpu.VMEM((SQ, HQ), jnp.float32),
            pltpu.VMEM((N_DEV, HQ, CHUNK, DH), jnp.float32),
            pltpu.VMEM((N_DEV, CHUNK, HQ), jnp.float32),
            pltpu.SemaphoreType.DMA((N_DEV,)),
            pltpu.SemaphoreType.DMA((N_DEV,)),
            pltpu.SemaphoreType.DMA((N_DEV,)),
            pltpu.SemaphoreType.DMA((N_DEV,)),
            pltpu.SemaphoreType.DMA((N_DEV,)),
            pltpu.SemaphoreType.DMA((N_DEV,)),
        ],
        compiler_params=pltpu.CompilerParams(
            collective_id=0, vmem_limit_bytes=64 * 1024 * 1024,
        ),
    )(
        x, Wq,
        K_ext.astype(jnp.bfloat16).reshape(1, SKV_LOCAL, HQ * DH),
        V_ext.astype(jnp.bfloat16).reshape(1, SKV_LOCAL, HQ * DH),
        Wo,
    )


# device time: 74342 ns/iter; 1.3400x vs baseline; 1.2474x over previous
import os

import jax
import jax.numpy as jnp
from jax import lax
from jax.experimental import pallas as pl
from jax.experimental.pallas import tpu as pltpu

_NO_COMM = os.environ.get("KERNEL_NO_COMM") == "1"

N_DEV = 8
SQ = 256
CHUNK = SQ // N_DEV
SKV_LOCAL = 4096
HQ = 8
DH = 128
D = 1024
BLK = 64
SCALE = 0.08838834764831843
NEG = -1e9


def kernel(x, Wq, K_ext, V_ext, Wo):
    def body(x_ref, wq_ref, k_ref, v_ref, wo_ref, out_ref,
             local_o, local_l, kcls, vcls, rs_o, rs_l,
             rs_send_o, rs_send_l, rs_recv_o, rs_recv_l,
             ag_send, ag_recv):
        my = lax.axis_index("i")
        h = pl.program_id(0)

        @pl.when(h == 0)
        def _():
            barrier = pltpu.get_barrier_semaphore()
            for p in range(N_DEV):
                pl.semaphore_signal(
                    barrier, inc=1,
                    device_id=(p,), device_id_type=pl.DeviceIdType.MESH,
                )
            pl.semaphore_wait(barrier, N_DEV)

        q_h = (jnp.dot(
            x_ref[0].astype(jnp.bfloat16),
            wq_ref[...].astype(jnp.bfloat16),
            preferred_element_type=jnp.float32,
        ) * SCALE).astype(jnp.bfloat16)

        k2d = k_ref[0]
        v2d = v_ref[0]

        @pl.when(h == 0)
        def _():
            z = jnp.zeros((BLK, DH), jnp.bfloat16)
            for c0 in (1, 2):
                kcls[c0, 21 * BLK:22 * BLK, :] = z
                vcls[c0, 21 * BLK:22 * BLK, :] = z

        for c0 in range(3):
            nblk = 22 if c0 == 0 else 21
            kcls[c0, 0:nblk * BLK, :] = jnp.concatenate(
                [k2d[(c0 + 3 * t) * BLK:(c0 + 3 * t + 1) * BLK]
                 for t in range(nblk)], axis=0)
            vcls[c0, 0:nblk * BLK, :] = jnp.concatenate(
                [v2d[(c0 + 3 * t) * BLK:(c0 + 3 * t + 1) * BLK]
                 for t in range(nblk)], axis=0)

        nact = 22 * BLK
        col = lax.broadcasted_iota(jnp.int32, (BLK, nact), 1)
        o_parts = []
        l_parts = []
        for qb in range(4):
            c = (2 * (qb + my)) % 3
            k_act = kcls[c]
            v_act = vcls[c]
            q_qb = q_h[qb * BLK:(qb + 1) * BLK]
            s = lax.dot_general(
                q_qb, k_act, (((1,), (1,)), ((), ())),
                preferred_element_type=jnp.float32,
            )
            w = jnp.exp(s.astype(jnp.bfloat16))
            w = jnp.where((col < 21 * BLK) | (c == 0), w, jnp.bfloat16(0))
            l_qb = jnp.sum(w.astype(jnp.float32), axis=1)
            o_qb = lax.dot_general(
                w, v_act, (((1,), (0,)), ((), ())),
                preferred_element_type=jnp.float32,
            )
            if qb in (1, 2):
                k_e = jnp.concatenate(
                    [k2d[0:BLK], k2d[qb * BLK:(qb + 1) * BLK]], axis=0)
                v_e = jnp.concatenate(
                    [v2d[0:BLK], v2d[qb * BLK:(qb + 1) * BLK]], axis=0)
                s_e = lax.dot_general(
                    q_qb, k_e, (((1,), (1,)), ((), ())),
                    preferred_element_type=jnp.float32,
                )
                w_e = jnp.exp(s_e.astype(jnp.bfloat16))
                w_e = jnp.where(my == 0, w_e, jnp.bfloat16(0))
                l_qb = l_qb + jnp.sum(w_e.astype(jnp.float32), axis=1)
                o_qb = o_qb + lax.dot_general(
                    w_e, v_e, (((1,), (0,)), ((), ())),
                    preferred_element_type=jnp.float32,
                )
            o_parts.append(o_qb)
            l_parts.append(l_qb)

        l_h = jnp.concatenate(l_parts)
        local_o[h] = jnp.concatenate(o_parts, axis=0)
        col = lax.broadcasted_iota(jnp.int32, (SQ, HQ), 1)
        local_l[...] = jnp.where(col == h, l_h[:, None], local_l[...])

        @pl.when(h == HQ - 1)
        def _():
            for p in range(N_DEV) if not _NO_COMM else []:
                @pl.when(my != p)
                def _():
                    d_o = pltpu.make_async_remote_copy(
                        src_ref=local_o.at[:, p * CHUNK:(p + 1) * CHUNK, :],
                        dst_ref=rs_o.at[my],
                        send_sem=rs_send_o.at[p], recv_sem=rs_recv_o.at[my],
                        device_id=(p,), device_id_type=pl.DeviceIdType.MESH,
                    )
                    d_l = pltpu.make_async_remote_copy(
                        src_ref=local_l.at[p * CHUNK:(p + 1) * CHUNK, :],
                        dst_ref=rs_l.at[my],
                        send_sem=rs_send_l.at[p], recv_sem=rs_recv_l.at[my],
                        device_id=(p,), device_id_type=pl.DeviceIdType.MESH,
                    )
                    d_o.start()
                    d_l.start()

            rs_o[my] = local_o[:, pl.ds(my * CHUNK, CHUNK), :]
            rs_l[my] = local_l[pl.ds(my * CHUNK, CHUNK), :]

            for p in range(N_DEV) if not _NO_COMM else []:
                @pl.when(my != p)
                def _():
                    r_o = pltpu.make_async_remote_copy(
                        src_ref=rs_o.at[p], dst_ref=rs_o.at[p],
                        send_sem=rs_send_o.at[p], recv_sem=rs_recv_o.at[p],
                        device_id=(p,), device_id_type=pl.DeviceIdType.MESH,
                    )
                    r_l = pltpu.make_async_remote_copy(
                        src_ref=rs_l.at[p], dst_ref=rs_l.at[p],
                        send_sem=rs_send_l.at[p], recv_sem=rs_recv_l.at[p],
                        device_id=(p,), device_id_type=pl.DeviceIdType.MESH,
                    )
                    r_o.wait_recv()
                    r_l.wait_recv()

            l_sum = jnp.sum(rs_l[...], axis=0)
            l_t = jnp.transpose(l_sum, (1, 0))
            o_acc = rs_o[0]
            for p in range(1, N_DEV):
                o_acc = o_acc + rs_o[p]
            ctx = o_acc / l_t[:, :, None]

            ctx2d = jnp.concatenate([ctx[i] for i in range(HQ)], axis=1)
            out_chunk = jnp.dot(
                ctx2d.astype(jnp.bfloat16),
                wo_ref[...].astype(jnp.bfloat16),
                preferred_element_type=jnp.float32,
            )
            out_ref[0, pl.ds(my * CHUNK, CHUNK), :] = out_chunk

            for p in range(N_DEV) if not _NO_COMM else []:
                @pl.when(my != p)
                def _():
                    d_g = pltpu.make_async_remote_copy(
                        src_ref=out_ref.at[0, pl.ds(my * CHUNK, CHUNK), :],
                        dst_ref=out_ref.at[0, pl.ds(my * CHUNK, CHUNK), :],
                        send_sem=ag_send.at[p], recv_sem=ag_recv.at[my],
                        device_id=(p,), device_id_type=pl.DeviceIdType.MESH,
                    )
                    d_g.start()
            for p in range(N_DEV) if not _NO_COMM else []:
                @pl.when(my != p)
                def _():
                    r_g = pltpu.make_async_remote_copy(
                        src_ref=out_ref.at[0, pl.ds(p * CHUNK, CHUNK), :],
                        dst_ref=out_ref.at[0, pl.ds(p * CHUNK, CHUNK), :],
                        send_sem=ag_send.at[p], recv_sem=ag_recv.at[p],
                        device_id=(p,), device_id_type=pl.DeviceIdType.MESH,
                    )
                    r_g.wait_recv()

            for p in range(N_DEV) if not _NO_COMM else []:
                @pl.when(my != p)
                def _():
                    w_o = pltpu.make_async_remote_copy(
                        src_ref=local_o.at[:, p * CHUNK:(p + 1) * CHUNK, :],
                        dst_ref=rs_o.at[my],
                        send_sem=rs_send_o.at[p], recv_sem=rs_recv_o.at[my],
                        device_id=(p,), device_id_type=pl.DeviceIdType.MESH,
                    )
                    w_l = pltpu.make_async_remote_copy(
                        src_ref=local_l.at[p * CHUNK:(p + 1) * CHUNK, :],
                        dst_ref=rs_l.at[my],
                        send_sem=rs_send_l.at[p], recv_sem=rs_recv_l.at[my],
                        device_id=(p,), device_id_type=pl.DeviceIdType.MESH,
                    )
                    w_g = pltpu.make_async_remote_copy(
                        src_ref=out_ref.at[0, pl.ds(my * CHUNK, CHUNK), :],
                        dst_ref=out_ref.at[0, pl.ds(my * CHUNK, CHUNK), :],
                        send_sem=ag_send.at[p], recv_sem=ag_recv.at[my],
                        device_id=(p,), device_id_type=pl.DeviceIdType.MESH,
                    )
                    w_o.wait_send()
                    w_l.wait_send()
                    w_g.wait_send()

    return pl.pallas_call(
        body,
        grid=(HQ,),
        out_shape=jax.ShapeDtypeStruct((1, SQ, D), jnp.float32),
        in_specs=[
            pl.BlockSpec((1, SQ, D), lambda h: (0, 0, 0)),
            pl.BlockSpec((D, DH), lambda h: (0, h)),
            pl.BlockSpec((1, SKV_LOCAL, DH), lambda h: (0, 0, h)),
            pl.BlockSpec((1, SKV_LOCAL, DH), lambda h: (0, 0, h)),
            pl.BlockSpec((D, D), lambda h: (0, 0)),
        ],
        out_specs=pl.BlockSpec((1, SQ, D), lambda h: (0, 0, 0)),
        scratch_shapes=[
            pltpu.VMEM((HQ, SQ, DH), jnp.float32),
            pltpu.VMEM((SQ, HQ), jnp.float32),
            pltpu.VMEM((3, 22 * BLK, DH), jnp.bfloat16),
            pltpu.VMEM((3, 22 * BLK, DH), jnp.bfloat16),
            pltpu.VMEM((N_DEV, HQ, CHUNK, DH), jnp.float32),
            pltpu.VMEM((N_DEV, CHUNK, HQ), jnp.float32),
            pltpu.SemaphoreType.DMA((N_DEV,)),
            pltpu.SemaphoreType.DMA((N_DEV,)),
            pltpu.SemaphoreType.DMA((N_DEV,)),
            pltpu.SemaphoreType.DMA((N_DEV,)),
            pltpu.SemaphoreType.DMA((N_DEV,)),
            pltpu.SemaphoreType.DMA((N_DEV,)),
        ],
        compiler_params=pltpu.CompilerParams(
            collective_id=0, vmem_limit_bytes=64 * 1024 * 1024,
        ),
    )(
        x, Wq,
        K_ext.astype(jnp.bfloat16).reshape(1, SKV_LOCAL, HQ * DH),
        V_ext.astype(jnp.bfloat16).reshape(1, SKV_LOCAL, HQ * DH),
        Wo,
    )


# device time: 70964 ns/iter; 1.4038x vs baseline; 1.0476x over previous
import os

import jax
import jax.numpy as jnp
from jax import lax
from jax.experimental import pallas as pl
from jax.experimental.pallas import tpu as pltpu

_NO_COMM = os.environ.get("KERNEL_NO_COMM") == "1"

N_DEV = 8
SQ = 256
CHUNK = SQ // N_DEV
SKV_LOCAL = 4096
HQ = 8
DH = 128
D = 1024
BLK = 64
SCALE = 0.08838834764831843
NEG = -1e9


def kernel(x, Wq, K_ext, V_ext, Wo):
    def body(x_ref, wq_ref, k_ref, v_ref, wo_ref, out_ref,
             local_o, local_l, kcls, vcls, rs_o, rs_l,
             rs_send_o, rs_send_l, rs_recv_o, rs_recv_l,
             ag_send, ag_recv):
        my = lax.axis_index("i")
        h = pl.program_id(0)

        @pl.when(h == 0)
        def _():
            barrier = pltpu.get_barrier_semaphore()
            for p in range(N_DEV):
                pl.semaphore_signal(
                    barrier, inc=1,
                    device_id=(p,), device_id_type=pl.DeviceIdType.MESH,
                )
            pl.semaphore_wait(barrier, N_DEV)

        q_h = (jnp.dot(
            x_ref[0].astype(jnp.bfloat16),
            wq_ref[...].astype(jnp.bfloat16),
            preferred_element_type=jnp.float32,
        ) * SCALE).astype(jnp.bfloat16)

        k2d = k_ref[0]
        v2d = v_ref[0]

        @pl.when(h == 0)
        def _():
            z = jnp.zeros((BLK, DH), jnp.bfloat16)
            for c0 in (1, 2):
                kcls[c0, 21 * BLK:22 * BLK, :] = z
                vcls[c0, 21 * BLK:22 * BLK, :] = z

        for c0 in range(3):
            nblk = 22 if c0 == 0 else 21
            kcls[c0, 0:nblk * BLK, :] = jnp.concatenate(
                [k2d[(c0 + 3 * t) * BLK:(c0 + 3 * t + 1) * BLK]
                 for t in range(nblk)], axis=0)
            vcls[c0, 0:nblk * BLK, :] = jnp.concatenate(
                [v2d[(c0 + 3 * t) * BLK:(c0 + 3 * t + 1) * BLK]
                 for t in range(nblk)], axis=0)

        nact = 22 * BLK
        col = lax.broadcasted_iota(jnp.int32, (BLK, nact), 1)
        o_parts = []
        l_parts = []
        for qb in range(4):
            c = (2 * (qb + my)) % 3
            k_act = kcls[c]
            v_act = vcls[c]
            q_qb = q_h[qb * BLK:(qb + 1) * BLK]
            s = lax.dot_general(
                q_qb, k_act, (((1,), (1,)), ((), ())),
                preferred_element_type=jnp.float32,
            )
            w = jnp.exp(s.astype(jnp.bfloat16))
            w = jnp.where((col < 21 * BLK) | (c == 0), w, jnp.bfloat16(0))
            l_qb = jnp.sum(w.astype(jnp.float32), axis=1)
            o_qb = lax.dot_general(
                w, v_act, (((1,), (0,)), ((), ())),
                preferred_element_type=jnp.float32,
            )
            if qb in (1, 2):
                k_e = jnp.concatenate(
                    [k2d[0:BLK], k2d[qb * BLK:(qb + 1) * BLK]], axis=0)
                v_e = jnp.concatenate(
                    [v2d[0:BLK], v2d[qb * BLK:(qb + 1) * BLK]], axis=0)
                s_e = lax.dot_general(
                    q_qb, k_e, (((1,), (1,)), ((), ())),
                    preferred_element_type=jnp.float32,
                )
                w_e = jnp.exp(s_e.astype(jnp.bfloat16))
                w_e = jnp.where(my == 0, w_e, jnp.bfloat16(0))
                l_qb = l_qb + jnp.sum(w_e.astype(jnp.float32), axis=1)
                o_qb = o_qb + lax.dot_general(
                    w_e, v_e, (((1,), (0,)), ((), ())),
                    preferred_element_type=jnp.float32,
                )
            o_parts.append(o_qb)
            l_parts.append(l_qb)

        l_h = jnp.concatenate(l_parts)
        local_o[h] = jnp.concatenate(o_parts, axis=0).astype(jnp.bfloat16)
        col = lax.broadcasted_iota(jnp.int32, (SQ, HQ), 1)
        local_l[...] = jnp.where(col == h, l_h[:, None], local_l[...])

        @pl.when(h == HQ - 1)
        def _():
            for p in range(N_DEV) if not _NO_COMM else []:
                @pl.when(my != p)
                def _():
                    d_o = pltpu.make_async_remote_copy(
                        src_ref=local_o.at[:, p * CHUNK:(p + 1) * CHUNK, :],
                        dst_ref=rs_o.at[my],
                        send_sem=rs_send_o.at[p], recv_sem=rs_recv_o.at[my],
                        device_id=(p,), device_id_type=pl.DeviceIdType.MESH,
                    )
                    d_l = pltpu.make_async_remote_copy(
                        src_ref=local_l.at[p * CHUNK:(p + 1) * CHUNK, :],
                        dst_ref=rs_l.at[my],
                        send_sem=rs_send_l.at[p], recv_sem=rs_recv_l.at[my],
                        device_id=(p,), device_id_type=pl.DeviceIdType.MESH,
                    )
                    d_o.start()
                    d_l.start()

            rs_o[my] = local_o[:, pl.ds(my * CHUNK, CHUNK), :]
            rs_l[my] = local_l[pl.ds(my * CHUNK, CHUNK), :]

            for p in range(N_DEV) if not _NO_COMM else []:
                @pl.when(my != p)
                def _():
                    r_o = pltpu.make_async_remote_copy(
                        src_ref=rs_o.at[p], dst_ref=rs_o.at[p],
                        send_sem=rs_send_o.at[p], recv_sem=rs_recv_o.at[p],
                        device_id=(p,), device_id_type=pl.DeviceIdType.MESH,
                    )
                    r_l = pltpu.make_async_remote_copy(
                        src_ref=rs_l.at[p], dst_ref=rs_l.at[p],
                        send_sem=rs_send_l.at[p], recv_sem=rs_recv_l.at[p],
                        device_id=(p,), device_id_type=pl.DeviceIdType.MESH,
                    )
                    r_o.wait_recv()
                    r_l.wait_recv()

            l_sum = jnp.sum(rs_l[...], axis=0)
            l_t = jnp.transpose(l_sum, (1, 0))
            o_acc = rs_o[0].astype(jnp.float32)
            for p in range(1, N_DEV):
                o_acc = o_acc + rs_o[p].astype(jnp.float32)
            ctx = o_acc / l_t[:, :, None]

            ctx2d = jnp.concatenate([ctx[i] for i in range(HQ)], axis=1)
            out_chunk = jnp.dot(
                ctx2d.astype(jnp.bfloat16),
                wo_ref[...].astype(jnp.bfloat16),
                preferred_element_type=jnp.float32,
            )
            out_ref[0, pl.ds(my * CHUNK, CHUNK), :] = out_chunk

            for p in range(N_DEV) if not _NO_COMM else []:
                @pl.when(my != p)
                def _():
                    d_g = pltpu.make_async_remote_copy(
                        src_ref=out_ref.at[0, pl.ds(my * CHUNK, CHUNK), :],
                        dst_ref=out_ref.at[0, pl.ds(my * CHUNK, CHUNK), :],
                        send_sem=ag_send.at[p], recv_sem=ag_recv.at[my],
                        device_id=(p,), device_id_type=pl.DeviceIdType.MESH,
                    )
                    d_g.start()
            for p in range(N_DEV) if not _NO_COMM else []:
                @pl.when(my != p)
                def _():
                    r_g = pltpu.make_async_remote_copy(
                        src_ref=out_ref.at[0, pl.ds(p * CHUNK, CHUNK), :],
                        dst_ref=out_ref.at[0, pl.ds(p * CHUNK, CHUNK), :],
                        send_sem=ag_send.at[p], recv_sem=ag_recv.at[p],
                        device_id=(p,), device_id_type=pl.DeviceIdType.MESH,
                    )
                    r_g.wait_recv()

            for p in range(N_DEV) if not _NO_COMM else []:
                @pl.when(my != p)
                def _():
                    w_o = pltpu.make_async_remote_copy(
                        src_ref=local_o.at[:, p * CHUNK:(p + 1) * CHUNK, :],
                        dst_ref=rs_o.at[my],
                        send_sem=rs_send_o.at[p], recv_sem=rs_recv_o.at[my],
                        device_id=(p,), device_id_type=pl.DeviceIdType.MESH,
                    )
                    w_l = pltpu.make_async_remote_copy(
                        src_ref=local_l.at[p * CHUNK:(p + 1) * CHUNK, :],
                        dst_ref=rs_l.at[my],
                        send_sem=rs_send_l.at[p], recv_sem=rs_recv_l.at[my],
                        device_id=(p,), device_id_type=pl.DeviceIdType.MESH,
                    )
                    w_g = pltpu.make_async_remote_copy(
                        src_ref=out_ref.at[0, pl.ds(my * CHUNK, CHUNK), :],
                        dst_ref=out_ref.at[0, pl.ds(my * CHUNK, CHUNK), :],
                        send_sem=ag_send.at[p], recv_sem=ag_recv.at[my],
                        device_id=(p,), device_id_type=pl.DeviceIdType.MESH,
                    )
                    w_o.wait_send()
                    w_l.wait_send()
                    w_g.wait_send()

    return pl.pallas_call(
        body,
        grid=(HQ,),
        out_shape=jax.ShapeDtypeStruct((1, SQ, D), jnp.float32),
        in_specs=[
            pl.BlockSpec((1, SQ, D), lambda h: (0, 0, 0)),
            pl.BlockSpec((D, DH), lambda h: (0, h)),
            pl.BlockSpec((1, SKV_LOCAL, DH), lambda h: (0, 0, h)),
            pl.BlockSpec((1, SKV_LOCAL, DH), lambda h: (0, 0, h)),
            pl.BlockSpec((D, D), lambda h: (0, 0)),
        ],
        out_specs=pl.BlockSpec((1, SQ, D), lambda h: (0, 0, 0)),
        scratch_shapes=[
            pltpu.VMEM((HQ, SQ, DH), jnp.bfloat16),
            pltpu.VMEM((SQ, HQ), jnp.float32),
            pltpu.VMEM((3, 22 * BLK, DH), jnp.bfloat16),
            pltpu.VMEM((3, 22 * BLK, DH), jnp.bfloat16),
            pltpu.VMEM((N_DEV, HQ, CHUNK, DH), jnp.bfloat16),
            pltpu.VMEM((N_DEV, CHUNK, HQ), jnp.float32),
            pltpu.SemaphoreType.DMA((N_DEV,)),
            pltpu.SemaphoreType.DMA((N_DEV,)),
            pltpu.SemaphoreType.DMA((N_DEV,)),
            pltpu.SemaphoreType.DMA((N_DEV,)),
            pltpu.SemaphoreType.DMA((N_DEV,)),
            pltpu.SemaphoreType.DMA((N_DEV,)),
        ],
        compiler_params=pltpu.CompilerParams(
            collective_id=0, vmem_limit_bytes=64 * 1024 * 1024,
        ),
    )(
        x, Wq,
        K_ext.astype(jnp.bfloat16).reshape(1, SKV_LOCAL, HQ * DH),
        V_ext.astype(jnp.bfloat16).reshape(1, SKV_LOCAL, HQ * DH),
        Wo,
    )


# device time: 69956 ns/iter; 1.4241x vs baseline; 1.0144x over previous
import os

import jax
import jax.numpy as jnp
from jax import lax
from jax.experimental import pallas as pl
from jax.experimental.pallas import tpu as pltpu

_NO_COMM = os.environ.get("KERNEL_NO_COMM") == "1"

N_DEV = 8
SQ = 256
CHUNK = SQ // N_DEV
SKV_LOCAL = 4096
HQ = 8
DH = 128
D = 1024
BLK = 64
SCALE = 0.08838834764831843
NEG = -1e9


def kernel(x, Wq, K_ext, V_ext, Wo):
    def body(x_ref, wq_ref, k_ref, v_ref, wo_ref, out_ref,
             local_o, local_l, kcls, vcls, rs_o, rs_l,
             rs_send_o, rs_send_l, rs_recv_o, rs_recv_l,
             ag_send, ag_recv):
        my = lax.axis_index("i")
        h = pl.program_id(0)

        @pl.when(h == 0)
        def _():
            barrier = pltpu.get_barrier_semaphore()
            for p in range(N_DEV):
                pl.semaphore_signal(
                    barrier, inc=1,
                    device_id=(p,), device_id_type=pl.DeviceIdType.MESH,
                )
            pl.semaphore_wait(barrier, N_DEV)

        q_h = jnp.dot(
            x_ref[0].astype(jnp.bfloat16),
            wq_ref[...].astype(jnp.bfloat16),
            preferred_element_type=jnp.float32,
        ).astype(jnp.float8_e4m3fn)

        k2d = k_ref[0]
        v2d = v_ref[0]

        @pl.when(h == 0)
        def _():
            zk = jnp.zeros((BLK, DH), jnp.float8_e4m3fn)
            zv = jnp.zeros((BLK, DH), jnp.bfloat16)
            for c0 in (1, 2):
                kcls[c0, 21 * BLK:22 * BLK, :] = zk
                vcls[c0, 21 * BLK:22 * BLK, :] = zv

        for c0 in range(3):
            nblk = 22 if c0 == 0 else 21
            kcls[c0, 0:nblk * BLK, :] = jnp.concatenate(
                [k2d[(c0 + 3 * t) * BLK:(c0 + 3 * t + 1) * BLK]
                 for t in range(nblk)], axis=0)
            vcls[c0, 0:nblk * BLK, :] = jnp.concatenate(
                [v2d[(c0 + 3 * t) * BLK:(c0 + 3 * t + 1) * BLK]
                 for t in range(nblk)], axis=0)

        nact = 22 * BLK
        col = lax.broadcasted_iota(jnp.int32, (BLK, nact), 1)
        o_parts = []
        l_parts = []
        for qb in range(4):
            c = (2 * (qb + my)) % 3
            k_act = kcls[c]
            v_act = vcls[c]
            q_qb = q_h[qb * BLK:(qb + 1) * BLK]
            s = lax.dot_general(
                q_qb, k_act, (((1,), (1,)), ((), ())),
                preferred_element_type=jnp.float32,
            )
            w = jnp.exp((s * SCALE).astype(jnp.bfloat16))
            w = jnp.where((col < 21 * BLK) | (c == 0), w, jnp.bfloat16(0))
            l_qb = jnp.sum(w.astype(jnp.float32), axis=1)
            o_qb = lax.dot_general(
                w, v_act, (((1,), (0,)), ((), ())),
                preferred_element_type=jnp.float32,
            )
            if qb in (1, 2):
                k_e = jnp.concatenate(
                    [k2d[0:BLK], k2d[qb * BLK:(qb + 1) * BLK]], axis=0)
                v_e = jnp.concatenate(
                    [v2d[0:BLK], v2d[qb * BLK:(qb + 1) * BLK]], axis=0)
                s_e = lax.dot_general(
                    q_qb, k_e, (((1,), (1,)), ((), ())),
                    preferred_element_type=jnp.float32,
                )
                w_e = jnp.exp((s_e * SCALE).astype(jnp.bfloat16))
                w_e = jnp.where(my == 0, w_e, jnp.bfloat16(0))
                l_qb = l_qb + jnp.sum(w_e.astype(jnp.float32), axis=1)
                o_qb = o_qb + lax.dot_general(
                    w_e, v_e, (((1,), (0,)), ((), ())),
                    preferred_element_type=jnp.float32,
                )
            o_parts.append(o_qb)
            l_parts.append(l_qb)

        l_h = jnp.concatenate(l_parts)
        local_o[h] = jnp.concatenate(o_parts, axis=0).astype(jnp.bfloat16)
        col = lax.broadcasted_iota(jnp.int32, (SQ, HQ), 1)
        local_l[...] = jnp.where(col == h, l_h[:, None], local_l[...])

        @pl.when(h == HQ - 1)
        def _():
            for p in range(N_DEV) if not _NO_COMM else []:
                @pl.when(my != p)
                def _():
                    d_o = pltpu.make_async_remote_copy(
                        src_ref=local_o.at[:, p * CHUNK:(p + 1) * CHUNK, :],
                        dst_ref=rs_o.at[my],
                        send_sem=rs_send_o.at[p], recv_sem=rs_recv_o.at[my],
                        device_id=(p,), device_id_type=pl.DeviceIdType.MESH,
                    )
                    d_l = pltpu.make_async_remote_copy(
                        src_ref=local_l.at[p * CHUNK:(p + 1) * CHUNK, :],
                        dst_ref=rs_l.at[my],
                        send_sem=rs_send_l.at[p], recv_sem=rs_recv_l.at[my],
                        device_id=(p,), device_id_type=pl.DeviceIdType.MESH,
                    )
                    d_o.start()
                    d_l.start()

            rs_o[my] = local_o[:, pl.ds(my * CHUNK, CHUNK), :]
            rs_l[my] = local_l[pl.ds(my * CHUNK, CHUNK), :]

            for p in range(N_DEV) if not _NO_COMM else []:
                @pl.when(my != p)
                def _():
                    r_o = pltpu.make_async_remote_copy(
                        src_ref=rs_o.at[p], dst_ref=rs_o.at[p],
                        send_sem=rs_send_o.at[p], recv_sem=rs_recv_o.at[p],
                        device_id=(p,), device_id_type=pl.DeviceIdType.MESH,
                    )
                    r_l = pltpu.make_async_remote_copy(
                        src_ref=rs_l.at[p], dst_ref=rs_l.at[p],
                        send_sem=rs_send_l.at[p], recv_sem=rs_recv_l.at[p],
                        device_id=(p,), device_id_type=pl.DeviceIdType.MESH,
                    )
                    r_o.wait_recv()
                    r_l.wait_recv()

            l_sum = jnp.sum(rs_l[...], axis=0)
            l_t = jnp.transpose(l_sum, (1, 0))
            o_acc = rs_o[0].astype(jnp.float32)
            for p in range(1, N_DEV):
                o_acc = o_acc + rs_o[p].astype(jnp.float32)
            ctx = o_acc / l_t[:, :, None]

            ctx2d = jnp.concatenate([ctx[i] for i in range(HQ)], axis=1)
            out_chunk = jnp.dot(
                ctx2d.astype(jnp.bfloat16),
                wo_ref[...].astype(jnp.bfloat16),
                preferred_element_type=jnp.float32,
            )
            out_ref[0, pl.ds(my * CHUNK, CHUNK), :] = out_chunk

            for p in range(N_DEV) if not _NO_COMM else []:
                @pl.when(my != p)
                def _():
                    d_g = pltpu.make_async_remote_copy(
                        src_ref=out_ref.at[0, pl.ds(my * CHUNK, CHUNK), :],
                        dst_ref=out_ref.at[0, pl.ds(my * CHUNK, CHUNK), :],
                        send_sem=ag_send.at[p], recv_sem=ag_recv.at[my],
                        device_id=(p,), device_id_type=pl.DeviceIdType.MESH,
                    )
                    d_g.start()
            for p in range(N_DEV) if not _NO_COMM else []:
                @pl.when(my != p)
                def _():
                    r_g = pltpu.make_async_remote_copy(
                        src_ref=out_ref.at[0, pl.ds(p * CHUNK, CHUNK), :],
                        dst_ref=out_ref.at[0, pl.ds(p * CHUNK, CHUNK), :],
                        send_sem=ag_send.at[p], recv_sem=ag_recv.at[p],
                        device_id=(p,), device_id_type=pl.DeviceIdType.MESH,
                    )
                    r_g.wait_recv()

            for p in range(N_DEV) if not _NO_COMM else []:
                @pl.when(my != p)
                def _():
                    w_o = pltpu.make_async_remote_copy(
                        src_ref=local_o.at[:, p * CHUNK:(p + 1) * CHUNK, :],
                        dst_ref=rs_o.at[my],
                        send_sem=rs_send_o.at[p], recv_sem=rs_recv_o.at[my],
                        device_id=(p,), device_id_type=pl.DeviceIdType.MESH,
                    )
                    w_l = pltpu.make_async_remote_copy(
                        src_ref=local_l.at[p * CHUNK:(p + 1) * CHUNK, :],
                        dst_ref=rs_l.at[my],
                        send_sem=rs_send_l.at[p], recv_sem=rs_recv_l.at[my],
                        device_id=(p,), device_id_type=pl.DeviceIdType.MESH,
                    )
                    w_g = pltpu.make_async_remote_copy(
                        src_ref=out_ref.at[0, pl.ds(my * CHUNK, CHUNK), :],
                        dst_ref=out_ref.at[0, pl.ds(my * CHUNK, CHUNK), :],
                        send_sem=ag_send.at[p], recv_sem=ag_recv.at[my],
                        device_id=(p,), device_id_type=pl.DeviceIdType.MESH,
                    )
                    w_o.wait_send()
                    w_l.wait_send()
                    w_g.wait_send()

    return pl.pallas_call(
        body,
        grid=(HQ,),
        out_shape=jax.ShapeDtypeStruct((1, SQ, D), jnp.float32),
        in_specs=[
            pl.BlockSpec((1, SQ, D), lambda h: (0, 0, 0)),
            pl.BlockSpec((D, DH), lambda h: (0, h)),
            pl.BlockSpec((1, SKV_LOCAL, DH), lambda h: (0, 0, h)),
            pl.BlockSpec((1, SKV_LOCAL, DH), lambda h: (0, 0, h)),
            pl.BlockSpec((D, D), lambda h: (0, 0)),
        ],
        out_specs=pl.BlockSpec((1, SQ, D), lambda h: (0, 0, 0)),
        scratch_shapes=[
            pltpu.VMEM((HQ, SQ, DH), jnp.bfloat16),
            pltpu.VMEM((SQ, HQ), jnp.float32),
            pltpu.VMEM((3, 22 * BLK, DH), jnp.float8_e4m3fn),
            pltpu.VMEM((3, 22 * BLK, DH), jnp.bfloat16),
            pltpu.VMEM((N_DEV, HQ, CHUNK, DH), jnp.bfloat16),
            pltpu.VMEM((N_DEV, CHUNK, HQ), jnp.float32),
            pltpu.SemaphoreType.DMA((N_DEV,)),
            pltpu.SemaphoreType.DMA((N_DEV,)),
            pltpu.SemaphoreType.DMA((N_DEV,)),
            pltpu.SemaphoreType.DMA((N_DEV,)),
            pltpu.SemaphoreType.DMA((N_DEV,)),
            pltpu.SemaphoreType.DMA((N_DEV,)),
        ],
        compiler_params=pltpu.CompilerParams(
            collective_id=0, vmem_limit_bytes=64 * 1024 * 1024,
        ),
    )(
        x, Wq,
        K_ext.astype(jnp.float8_e4m3fn).reshape(1, SKV_LOCAL, HQ * DH),
        V_ext.astype(jnp.bfloat16).reshape(1, SKV_LOCAL, HQ * DH),
        Wo,
    )


# device time: 67922 ns/iter; 1.4667x vs baseline; 1.0299x over previous
import os

import jax
import jax.numpy as jnp
from jax import lax
from jax.experimental import pallas as pl
from jax.experimental.pallas import tpu as pltpu

_NO_COMM = os.environ.get("KERNEL_NO_COMM") == "1"

N_DEV = 8
SQ = 256
CHUNK = SQ // N_DEV
SKV_LOCAL = 4096
HQ = 8
DH = 128
D = 1024
BLK = 64
SCALE = 0.08838834764831843
NEG = -1e9


def kernel(x, Wq, K_ext, V_ext, Wo):
    def body(x_ref, wq_ref, k_ref, v_ref, wo_ref, out_ref,
             local_o, local_l, kcls, vcls, rs_o, rs_l, ag_buf,
             rs_send_o, rs_send_l, rs_recv_o, rs_recv_l,
             ag_send, ag_recv):
        my = lax.axis_index("i")
        h = pl.program_id(0)

        @pl.when(h == 0)
        def _():
            barrier = pltpu.get_barrier_semaphore()
            for p in range(N_DEV):
                pl.semaphore_signal(
                    barrier, inc=1,
                    device_id=(p,), device_id_type=pl.DeviceIdType.MESH,
                )
            pl.semaphore_wait(barrier, N_DEV)

        q_h = (jnp.dot(
            x_ref[0].astype(jnp.bfloat16),
            wq_ref[...].astype(jnp.bfloat16),
            preferred_element_type=jnp.float32,
        ) * SCALE).astype(jnp.bfloat16)

        k2d = k_ref[0]
        v2d = v_ref[0]

        @pl.when(h == 0)
        def _():
            z = jnp.zeros((BLK, DH), jnp.bfloat16)
            for c0 in (1, 2):
                kcls[c0, 21 * BLK:22 * BLK, :] = z
                vcls[c0, 21 * BLK:22 * BLK, :] = z

        for c0 in range(3):
            nblk = 22 if c0 == 0 else 21
            kcls[c0, 0:nblk * BLK, :] = jnp.concatenate(
                [k2d[(c0 + 3 * t) * BLK:(c0 + 3 * t + 1) * BLK]
                 for t in range(nblk)], axis=0)
            vcls[c0, 0:nblk * BLK, :] = jnp.concatenate(
                [v2d[(c0 + 3 * t) * BLK:(c0 + 3 * t + 1) * BLK]
                 for t in range(nblk)], axis=0)

        nact = 22 * BLK
        col = lax.broadcasted_iota(jnp.int32, (BLK, nact), 1)
        o_parts = []
        l_parts = []
        for qb in range(4):
            c = (2 * (qb + my)) % 3
            k_act = kcls[c]
            v_act = vcls[c]
            q_qb = q_h[qb * BLK:(qb + 1) * BLK]
            s = lax.dot_general(
                q_qb, k_act, (((1,), (1,)), ((), ())),
                preferred_element_type=jnp.float32,
            )
            w = jnp.exp(s.astype(jnp.bfloat16))
            w = jnp.where((col < 21 * BLK) | (c == 0), w, jnp.bfloat16(0))
            l_qb = jnp.sum(w.astype(jnp.float32), axis=1)
            o_qb = lax.dot_general(
                w, v_act, (((1,), (0,)), ((), ())),
                preferred_element_type=jnp.float32,
            )
            if qb in (1, 2):
                k_e = jnp.concatenate(
                    [k2d[0:BLK], k2d[qb * BLK:(qb + 1) * BLK]], axis=0)
                v_e = jnp.concatenate(
                    [v2d[0:BLK], v2d[qb * BLK:(qb + 1) * BLK]], axis=0)
                s_e = lax.dot_general(
                    q_qb, k_e, (((1,), (1,)), ((), ())),
                    preferred_element_type=jnp.float32,
                )
                w_e = jnp.exp(s_e.astype(jnp.bfloat16))
                w_e = jnp.where(my == 0, w_e, jnp.bfloat16(0))
                l_qb = l_qb + jnp.sum(w_e.astype(jnp.float32), axis=1)
                o_qb = o_qb + lax.dot_general(
                    w_e, v_e, (((1,), (0,)), ((), ())),
                    preferred_element_type=jnp.float32,
                )
            o_parts.append(o_qb)
            l_parts.append(l_qb)

        l_h = jnp.concatenate(l_parts)
        local_o[h] = jnp.concatenate(o_parts, axis=0).astype(jnp.bfloat16)
        col = lax.broadcasted_iota(jnp.int32, (SQ, HQ), 1)
        local_l[...] = jnp.where(col == h, l_h[:, None], local_l[...])

        @pl.when(h == HQ - 1)
        def _():
            for p in range(N_DEV) if not _NO_COMM else []:
                @pl.when(my != p)
                def _():
                    d_o = pltpu.make_async_remote_copy(
                        src_ref=local_o.at[:, p * CHUNK:(p + 1) * CHUNK, :],
                        dst_ref=rs_o.at[my],
                        send_sem=rs_send_o.at[p], recv_sem=rs_recv_o.at[my],
                        device_id=(p,), device_id_type=pl.DeviceIdType.MESH,
                    )
                    d_l = pltpu.make_async_remote_copy(
                        src_ref=local_l.at[p * CHUNK:(p + 1) * CHUNK, :],
                        dst_ref=rs_l.at[my],
                        send_sem=rs_send_l.at[p], recv_sem=rs_recv_l.at[my],
                        device_id=(p,), device_id_type=pl.DeviceIdType.MESH,
                    )
                    d_o.start()
                    d_l.start()

            rs_o[my] = local_o[:, pl.ds(my * CHUNK, CHUNK), :]
            rs_l[my] = local_l[pl.ds(my * CHUNK, CHUNK), :]

            for p in range(N_DEV) if not _NO_COMM else []:
                @pl.when(my != p)
                def _():
                    r_o = pltpu.make_async_remote_copy(
                        src_ref=rs_o.at[p], dst_ref=rs_o.at[p],
                        send_sem=rs_send_o.at[p], recv_sem=rs_recv_o.at[p],
                        device_id=(p,), device_id_type=pl.DeviceIdType.MESH,
                    )
                    r_l = pltpu.make_async_remote_copy(
                        src_ref=rs_l.at[p], dst_ref=rs_l.at[p],
                        send_sem=rs_send_l.at[p], recv_sem=rs_recv_l.at[p],
                        device_id=(p,), device_id_type=pl.DeviceIdType.MESH,
                    )
                    r_o.wait_recv()
                    r_l.wait_recv()

            l_sum = jnp.sum(rs_l[...], axis=0)
            l_t = jnp.transpose(l_sum, (1, 0))
            o_acc = rs_o[0].astype(jnp.float32)
            for p in range(1, N_DEV):
                o_acc = o_acc + rs_o[p].astype(jnp.float32)
            ctx = o_acc / l_t[:, :, None]

            ctx2d = jnp.concatenate([ctx[i] for i in range(HQ)], axis=1)
            out_chunk = jnp.dot(
                ctx2d.astype(jnp.bfloat16),
                wo_ref[...].astype(jnp.bfloat16),
                preferred_element_type=jnp.float32,
            )
            out_ref[0, pl.ds(my * CHUNK, CHUNK), :] = out_chunk
            ag_buf[my] = out_chunk.astype(jnp.bfloat16)

            for p in range(N_DEV) if not _NO_COMM else []:
                @pl.when(my != p)
                def _():
                    d_g = pltpu.make_async_remote_copy(
                        src_ref=ag_buf.at[my], dst_ref=ag_buf.at[my],
                        send_sem=ag_send.at[p], recv_sem=ag_recv.at[my],
                        device_id=(p,), device_id_type=pl.DeviceIdType.MESH,
                    )
                    d_g.start()
            for p in range(N_DEV) if not _NO_COMM else []:
                @pl.when(my != p)
                def _():
                    r_g = pltpu.make_async_remote_copy(
                        src_ref=ag_buf.at[p], dst_ref=ag_buf.at[p],
                        send_sem=ag_send.at[p], recv_sem=ag_recv.at[p],
                        device_id=(p,), device_id_type=pl.DeviceIdType.MESH,
                    )
                    r_g.wait_recv()
                    out_ref[0, p * CHUNK:(p + 1) * CHUNK, :] = (
                        ag_buf[p].astype(jnp.float32))

            for p in range(N_DEV) if not _NO_COMM else []:
                @pl.when(my != p)
                def _():
                    w_o = pltpu.make_async_remote_copy(
                        src_ref=local_o.at[:, p * CHUNK:(p + 1) * CHUNK, :],
                        dst_ref=rs_o.at[my],
                        send_sem=rs_send_o.at[p], recv_sem=rs_recv_o.at[my],
                        device_id=(p,), device_id_type=pl.DeviceIdType.MESH,
                    )
                    w_l = pltpu.make_async_remote_copy(
                        src_ref=local_l.at[p * CHUNK:(p + 1) * CHUNK, :],
                        dst_ref=rs_l.at[my],
                        send_sem=rs_send_l.at[p], recv_sem=rs_recv_l.at[my],
                        device_id=(p,), device_id_type=pl.DeviceIdType.MESH,
                    )
                    w_g = pltpu.make_async_remote_copy(
                        src_ref=ag_buf.at[my], dst_ref=ag_buf.at[my],
                        send_sem=ag_send.at[p], recv_sem=ag_recv.at[my],
                        device_id=(p,), device_id_type=pl.DeviceIdType.MESH,
                    )
                    w_o.wait_send()
                    w_l.wait_send()
                    w_g.wait_send()

    return pl.pallas_call(
        body,
        grid=(HQ,),
        out_shape=jax.ShapeDtypeStruct((1, SQ, D), jnp.float32),
        in_specs=[
            pl.BlockSpec((1, SQ, D), lambda h: (0, 0, 0)),
            pl.BlockSpec((D, DH), lambda h: (0, h)),
            pl.BlockSpec((1, SKV_LOCAL, DH), lambda h: (0, 0, h)),
            pl.BlockSpec((1, SKV_LOCAL, DH), lambda h: (0, 0, h)),
            pl.BlockSpec((D, D), lambda h: (0, 0)),
        ],
        out_specs=pl.BlockSpec((1, SQ, D), lambda h: (0, 0, 0)),
        scratch_shapes=[
            pltpu.VMEM((HQ, SQ, DH), jnp.bfloat16),
            pltpu.VMEM((SQ, HQ), jnp.float32),
            pltpu.VMEM((3, 22 * BLK, DH), jnp.bfloat16),
            pltpu.VMEM((3, 22 * BLK, DH), jnp.bfloat16),
            pltpu.VMEM((N_DEV, HQ, CHUNK, DH), jnp.bfloat16),
            pltpu.VMEM((N_DEV, CHUNK, HQ), jnp.float32),
            pltpu.VMEM((N_DEV, CHUNK, D), jnp.bfloat16),
            pltpu.SemaphoreType.DMA((N_DEV,)),
            pltpu.SemaphoreType.DMA((N_DEV,)),
            pltpu.SemaphoreType.DMA((N_DEV,)),
            pltpu.SemaphoreType.DMA((N_DEV,)),
            pltpu.SemaphoreType.DMA((N_DEV,)),
            pltpu.SemaphoreType.DMA((N_DEV,)),
        ],
        compiler_params=pltpu.CompilerParams(
            collective_id=0, vmem_limit_bytes=64 * 1024 * 1024,
        ),
    )(
        x, Wq,
        K_ext.astype(jnp.bfloat16).reshape(1, SKV_LOCAL, HQ * DH),
        V_ext.astype(jnp.bfloat16).reshape(1, SKV_LOCAL, HQ * DH),
        Wo,
    )


# device time: 66348 ns/iter; 1.5015x vs baseline; 1.0237x over previous
import os

import jax
import jax.numpy as jnp
from jax import lax
from jax.experimental import pallas as pl
from jax.experimental.pallas import tpu as pltpu

_NO_COMM = os.environ.get("KERNEL_NO_COMM") == "1"

N_DEV = 8
SQ = 256
CHUNK = SQ // N_DEV
SKV_LOCAL = 4096
HQ = 8
DH = 128
D = 1024
BLK = 64
SCALE = 0.08838834764831843
NEG = -1e9


def _retile(kv):
    seg = 512

    def rbody(in_ref, out_ref):
        out_ref[0] = in_ref[0].reshape(seg, HQ * DH).astype(jnp.bfloat16)

    return pl.pallas_call(
        rbody,
        grid=(SKV_LOCAL // seg,),
        out_shape=jax.ShapeDtypeStruct((1, SKV_LOCAL, HQ * DH), jnp.bfloat16),
        in_specs=[pl.BlockSpec((1, seg, HQ, DH), lambda g: (0, g, 0, 0))],
        out_specs=pl.BlockSpec((1, seg, HQ * DH), lambda g: (0, g, 0)),
    )(kv)


def kernel(x, Wq, K_ext, V_ext, Wo):
    def body(x_ref, wq_ref, k_ref, v_ref, wo_ref, out_ref,
             local_o, local_l, kcls, vcls, rs_o, rs_l, ag_buf,
             rs_send_o, rs_send_l, rs_recv_o, rs_recv_l,
             ag_send, ag_recv):
        my = lax.axis_index("i")
        h = pl.program_id(0)

        @pl.when(h == 0)
        def _():
            barrier = pltpu.get_barrier_semaphore()
            for p in range(N_DEV):
                pl.semaphore_signal(
                    barrier, inc=1,
                    device_id=(p,), device_id_type=pl.DeviceIdType.MESH,
                )
            pl.semaphore_wait(barrier, N_DEV)

        q_h = (jnp.dot(
            x_ref[0].astype(jnp.bfloat16),
            wq_ref[...].astype(jnp.bfloat16),
            preferred_element_type=jnp.float32,
        ) * SCALE).astype(jnp.bfloat16)

        k2d = k_ref[0]
        v2d = v_ref[0]

        @pl.when(h == 0)
        def _():
            z = jnp.zeros((BLK, DH), jnp.bfloat16)
            for c0 in (1, 2):
                kcls[c0, 21 * BLK:22 * BLK, :] = z
                vcls[c0, 21 * BLK:22 * BLK, :] = z

        for c0 in range(3):
            nblk = 22 if c0 == 0 else 21
            kcls[c0, 0:nblk * BLK, :] = jnp.concatenate(
                [k2d[(c0 + 3 * t) * BLK:(c0 + 3 * t + 1) * BLK]
                 for t in range(nblk)], axis=0)
            vcls[c0, 0:nblk * BLK, :] = jnp.concatenate(
                [v2d[(c0 + 3 * t) * BLK:(c0 + 3 * t + 1) * BLK]
                 for t in range(nblk)], axis=0)

        nact = 22 * BLK
        col = lax.broadcasted_iota(jnp.int32, (BLK, nact), 1)
        o_parts = []
        l_parts = []
        for qb in range(4):
            c = (2 * (qb + my)) % 3
            k_act = kcls[c]
            v_act = vcls[c]
            q_qb = q_h[qb * BLK:(qb + 1) * BLK]
            s = lax.dot_general(
                q_qb, k_act, (((1,), (1,)), ((), ())),
                preferred_element_type=jnp.float32,
            )
            w = jnp.exp(s.astype(jnp.bfloat16))
            w = jnp.where((col < 21 * BLK) | (c == 0), w, jnp.bfloat16(0))
            l_qb = jnp.sum(w.astype(jnp.float32), axis=1)
            o_qb = lax.dot_general(
                w, v_act, (((1,), (0,)), ((), ())),
                preferred_element_type=jnp.float32,
            )
            if qb in (1, 2):
                k_e = jnp.concatenate(
                    [k2d[0:BLK], k2d[qb * BLK:(qb + 1) * BLK]], axis=0)
                v_e = jnp.concatenate(
                    [v2d[0:BLK], v2d[qb * BLK:(qb + 1) * BLK]], axis=0)
                s_e = lax.dot_general(
                    q_qb, k_e, (((1,), (1,)), ((), ())),
                    preferred_element_type=jnp.float32,
                )
                w_e = jnp.exp(s_e.astype(jnp.bfloat16))
                w_e = jnp.where(my == 0, w_e, jnp.bfloat16(0))
                l_qb = l_qb + jnp.sum(w_e.astype(jnp.float32), axis=1)
                o_qb = o_qb + lax.dot_general(
                    w_e, v_e, (((1,), (0,)), ((), ())),
                    preferred_element_type=jnp.float32,
                )
            o_parts.append(o_qb)
            l_parts.append(l_qb)

        l_h = jnp.concatenate(l_parts)
        local_o[h] = jnp.concatenate(o_parts, axis=0).astype(jnp.bfloat16)
        col = lax.broadcasted_iota(jnp.int32, (SQ, HQ), 1)
        local_l[...] = jnp.where(col == h, l_h[:, None], local_l[...])

        @pl.when(h == HQ - 1)
        def _():
            for p in range(N_DEV) if not _NO_COMM else []:
                @pl.when(my != p)
                def _():
                    d_o = pltpu.make_async_remote_copy(
                        src_ref=local_o.at[:, p * CHUNK:(p + 1) * CHUNK, :],
                        dst_ref=rs_o.at[my],
                        send_sem=rs_send_o.at[p], recv_sem=rs_recv_o.at[my],
                        device_id=(p,), device_id_type=pl.DeviceIdType.MESH,
                    )
                    d_l = pltpu.make_async_remote_copy(
                        src_ref=local_l.at[p * CHUNK:(p + 1) * CHUNK, :],
                        dst_ref=rs_l.at[my],
                        send_sem=rs_send_l.at[p], recv_sem=rs_recv_l.at[my],
                        device_id=(p,), device_id_type=pl.DeviceIdType.MESH,
                    )
                    d_o.start()
                    d_l.start()

            rs_o[my] = local_o[:, pl.ds(my * CHUNK, CHUNK), :]
            rs_l[my] = local_l[pl.ds(my * CHUNK, CHUNK), :]

            for p in range(N_DEV) if not _NO_COMM else []:
                @pl.when(my != p)
                def _():
                    r_o = pltpu.make_async_remote_copy(
                        src_ref=rs_o.at[p], dst_ref=rs_o.at[p],
                        send_sem=rs_send_o.at[p], recv_sem=rs_recv_o.at[p],
                        device_id=(p,), device_id_type=pl.DeviceIdType.MESH,
                    )
                    r_l = pltpu.make_async_remote_copy(
                        src_ref=rs_l.at[p], dst_ref=rs_l.at[p],
                        send_sem=rs_send_l.at[p], recv_sem=rs_recv_l.at[p],
                        device_id=(p,), device_id_type=pl.DeviceIdType.MESH,
                    )
                    r_o.wait_recv()
                    r_l.wait_recv()

            l_sum = jnp.sum(rs_l[...], axis=0)
            l_t = jnp.transpose(l_sum, (1, 0))
            o_acc = rs_o[0].astype(jnp.float32)
            for p in range(1, N_DEV):
                o_acc = o_acc + rs_o[p].astype(jnp.float32)
            ctx = o_acc / l_t[:, :, None]

            ctx2d = jnp.concatenate([ctx[i] for i in range(HQ)], axis=1)
            out_chunk = jnp.dot(
                ctx2d.astype(jnp.bfloat16),
                wo_ref[...].astype(jnp.bfloat16),
                preferred_element_type=jnp.float32,
            )
            out_ref[0, pl.ds(my * CHUNK, CHUNK), :] = out_chunk
            ag_buf[my] = out_chunk.astype(jnp.bfloat16)

            for p in range(N_DEV) if not _NO_COMM else []:
                @pl.when(my != p)
                def _():
                    d_g = pltpu.make_async_remote_copy(
                        src_ref=ag_buf.at[my], dst_ref=ag_buf.at[my],
                        send_sem=ag_send.at[p], recv_sem=ag_recv.at[my],
                        device_id=(p,), device_id_type=pl.DeviceIdType.MESH,
                    )
                    d_g.start()
            for p in range(N_DEV) if not _NO_COMM else []:
                @pl.when(my != p)
                def _():
                    r_g = pltpu.make_async_remote_copy(
                        src_ref=ag_buf.at[p], dst_ref=ag_buf.at[p],
                        send_sem=ag_send.at[p], recv_sem=ag_recv.at[p],
                        device_id=(p,), device_id_type=pl.DeviceIdType.MESH,
                    )
                    r_g.wait_recv()
                    out_ref[0, p * CHUNK:(p + 1) * CHUNK, :] = (
                        ag_buf[p].astype(jnp.float32))

            for p in range(N_DEV) if not _NO_COMM else []:
                @pl.when(my != p)
                def _():
                    w_o = pltpu.make_async_remote_copy(
                        src_ref=local_o.at[:, p * CHUNK:(p + 1) * CHUNK, :],
                        dst_ref=rs_o.at[my],
                        send_sem=rs_send_o.at[p], recv_sem=rs_recv_o.at[my],
                        device_id=(p,), device_id_type=pl.DeviceIdType.MESH,
                    )
                    w_l = pltpu.make_async_remote_copy(
                        src_ref=local_l.at[p * CHUNK:(p + 1) * CHUNK, :],
                        dst_ref=rs_l.at[my],
                        send_sem=rs_send_l.at[p], recv_sem=rs_recv_l.at[my],
                        device_id=(p,), device_id_type=pl.DeviceIdType.MESH,
                    )
                    w_g = pltpu.make_async_remote_copy(
                        src_ref=ag_buf.at[my], dst_ref=ag_buf.at[my],
                        send_sem=ag_send.at[p], recv_sem=ag_recv.at[my],
                        device_id=(p,), device_id_type=pl.DeviceIdType.MESH,
                    )
                    w_o.wait_send()
                    w_l.wait_send()
                    w_g.wait_send()

    return pl.pallas_call(
        body,
        grid=(HQ,),
        out_shape=jax.ShapeDtypeStruct((1, SQ, D), jnp.float32),
        in_specs=[
            pl.BlockSpec((1, SQ, D), lambda h: (0, 0, 0)),
            pl.BlockSpec((D, DH), lambda h: (0, h)),
            pl.BlockSpec((1, SKV_LOCAL, DH), lambda h: (0, 0, h)),
            pl.BlockSpec((1, SKV_LOCAL, DH), lambda h: (0, 0, h)),
            pl.BlockSpec((D, D), lambda h: (0, 0)),
        ],
        out_specs=pl.BlockSpec((1, SQ, D), lambda h: (0, 0, 0)),
        scratch_shapes=[
            pltpu.VMEM((HQ, SQ, DH), jnp.bfloat16),
            pltpu.VMEM((SQ, HQ), jnp.float32),
            pltpu.VMEM((3, 22 * BLK, DH), jnp.bfloat16),
            pltpu.VMEM((3, 22 * BLK, DH), jnp.bfloat16),
            pltpu.VMEM((N_DEV, HQ, CHUNK, DH), jnp.bfloat16),
            pltpu.VMEM((N_DEV, CHUNK, HQ), jnp.float32),
            pltpu.VMEM((N_DEV, CHUNK, D), jnp.bfloat16),
            pltpu.SemaphoreType.DMA((N_DEV,)),
            pltpu.SemaphoreType.DMA((N_DEV,)),
            pltpu.SemaphoreType.DMA((N_DEV,)),
            pltpu.SemaphoreType.DMA((N_DEV,)),
            pltpu.SemaphoreType.DMA((N_DEV,)),
            pltpu.SemaphoreType.DMA((N_DEV,)),
        ],
        compiler_params=pltpu.CompilerParams(
            collective_id=0, vmem_limit_bytes=64 * 1024 * 1024,
        ),
    )(
        x, Wq,
        _retile(K_ext),
        _retile(V_ext),
        Wo,
    )
